# baseline (speedup 1.0000x reference)
import functools

import jax
import jax.numpy as jnp
from jax import lax
from jax.experimental import pallas as pl
from jax.experimental.pallas import tpu as pltpu
from jax.experimental.pallas import tpu_sc as plsc

D_MODEL = 32
_ROWS, _COLS = 16384, 26
_L = 128
_RPU = 32
_U = _RPU * _COLS
_NW = 32
_NU = _ROWS // _RPU // _NW
_NSTREAM = -(-_U // _L)
_NVEC = _U // 16


def _make_gather():
    mesh = plsc.VectorSubcoreMesh(core_axis_name="c", subcore_axis_name="s")

    @functools.partial(
        pl.kernel,
        out_type=jax.ShapeDtypeStruct((_ROWS, _COLS, D_MODEL), jnp.float32),
        mesh=mesh,
        scratch_types=[
            pltpu.VMEM((2, _RPU, _COLS), jnp.float32),
            pltpu.VMEM((2, _U), jnp.int32),
            pltpu.VMEM((2, _U, D_MODEL), jnp.float32),
            pltpu.SemaphoreType.DMA,
            pltpu.SemaphoreType.DMA,
            pltpu.SemaphoreType.DMA,
            pltpu.SemaphoreType.DMA,
        ],
        compiler_params=pltpu.CompilerParams(use_tc_tiling_on_sc=False,
                                             needs_layout_passes=False),
    )
    def gather(table_hbm, x_hbm, out_hbm, xblk_v, idx_v, rows_v,
               gsem0, gsem1, osem0, osem1):
        wid = lax.axis_index("s") * 2 + lax.axis_index("c")
        ubase = wid * _NU
        gsems = (gsem0, gsem1)
        osems = (osem0, osem1)

        def stream_len(k):
            return min(_L, _U - k * _L)

        @pl.loop(0, _NU, step=2)
        def _pair(uo):
            for b in range(2):
                i0 = (ubase + uo + b) * _RPU

                @pl.when(uo != 0)
                def _reclaim():
                    for r in range(_RPU):
                        pltpu.make_async_copy(
                            rows_v.at[b, pl.ds(r * _COLS, _COLS)],
                            out_hbm.at[i0 + r],
                            osems[b]).wait()

                pltpu.sync_copy(x_hbm.at[pl.ds(i0, _RPU)], xblk_v.at[b])
                for v in range(_NVEC):
                    p = lax.iota(jnp.int32, 16) + (v * 16)
                    vals = plsc.load_gather(xblk_v.at[b],
                                            [p // _COLS, p % _COLS])
                    idx_v[b, pl.ds(v * 16, 16)] = plsc.bitcast(
                        vals, jnp.int32)
                for k in range(_NSTREAM):
                    n = stream_len(k)
                    pltpu.async_copy(
                        table_hbm.at[idx_v.at[b, pl.ds(k * _L, n)]],
                        rows_v.at[b, pl.ds(k * _L, n)],
                        gsems[b])
            for b in range(2):
                i0 = (ubase + uo + b) * _RPU
                for k in range(_NSTREAM):
                    n = stream_len(k)
                    pltpu.make_async_copy(
                        table_hbm.at[idx_v.at[b, pl.ds(k * _L, n)]],
                        rows_v.at[b, pl.ds(k * _L, n)],
                        gsems[b]).wait()
                for r in range(_RPU):
                    pltpu.async_copy(rows_v.at[b, pl.ds(r * _COLS, _COLS)],
                                     out_hbm.at[i0 + r],
                                     osems[b])

        for b in range(2):
            i0 = (ubase + _NU - 2 + b) * _RPU
            for r in range(_RPU):
                pltpu.make_async_copy(
                    rows_v.at[b, pl.ds(r * _COLS, _COLS)],
                    out_hbm.at[i0 + r],
                    osems[b]).wait()

    return gather


_gather = _make_gather()


@jax.jit
def kernel(x, embed_weight):
    xf = lax.bitcast_convert_type(x.astype(jnp.int32), jnp.float32)
    return _gather(embed_weight, xf)
